# Initial kernel scaffold; baseline (speedup 1.0000x reference)
import jax
import jax.numpy as jnp
from jax.experimental import pallas as pl


def kernel(input1, input2, eps, iters):
    raise NotImplementedError("write your pallas kernel here")


# speedup vs baseline: 28.1479x; 28.1479x over previous
import jax
import jax.numpy as jnp
from jax import lax
from jax.experimental import pallas as pl
from jax.experimental.pallas import tpu as pltpu

_ROWS_PER_BLOCK = 256
_NEG_INF = float("-inf")
_POS_INF = float("inf")


def _auction_kernel(x1_ref, x2t_ref, eps_ref, iters_ref, dist_ref, asg_out_ref,
                    asg_ref, j1s_ref, bids_ref, price_ref, maxinc_ref,
                    winner_ref):
    n = x1_ref.shape[1]
    m = x2t_ref.shape[2]
    R = _ROWS_PER_BLOCK
    nb = n // R
    eps = eps_ref[0]
    iters = iters_ref[0]

    lane = lax.broadcasted_iota(jnp.int32, (R, m), 1)

    def cost_block(r):
        x1b = x1_ref[0, pl.ds(r * R, R), :]
        c = None
        for d in range(3):
            dd = x1b[:, d:d + 1] - x2t_ref[0, d:d + 1, :]
            sq = dd * dd
            c = sq if c is None else c + sq
        return c

    def rows_iota(r):
        return r * R + lax.broadcasted_iota(jnp.int32, (R, 1), 0)

    def pass_bid(r, carry):
        total = cost_block(r) + price_ref[...]
        v1 = jnp.min(total, axis=1, keepdims=True)
        j1 = jnp.min(jnp.where(total == v1, lane, m), axis=1, keepdims=True)
        v2 = jnp.min(jnp.where(lane == j1, _POS_INF, total), axis=1,
                     keepdims=True)
        asgb = asg_ref[pl.ds(r * R, R), :]
        bid = jnp.where(asgb < 0, v2 - v1 + eps, _NEG_INF)
        j1s_ref[pl.ds(r * R, R), :] = j1
        bids_ref[pl.ds(r * R, R), :] = bid
        onehot = lane == j1
        contrib = jnp.max(jnp.where(onehot, bid, _NEG_INF), axis=0,
                          keepdims=True)
        maxinc_ref[...] = jnp.maximum(maxinc_ref[...], contrib)
        return carry

    def pass_winner(r, carry):
        j1b = j1s_ref[pl.ds(r * R, R), :]
        bidb = bids_ref[pl.ds(r * R, R), :]
        asgb = asg_ref[pl.ds(r * R, R), :]
        onehot = lane == j1b
        gmax = jnp.max(jnp.where(onehot, maxinc_ref[...], _NEG_INF), axis=1,
                       keepdims=True)
        cand = jnp.logical_and(asgb < 0, bidb >= gmax)
        contrib = jnp.max(
            jnp.where(jnp.logical_and(onehot, cand), rows_iota(r), -1),
            axis=0, keepdims=True)
        winner_ref[...] = jnp.maximum(winner_ref[...], contrib)
        return carry

    def make_pass_update(active):
        def pass_update(r, carry):
            j1b = j1s_ref[pl.ds(r * R, R), :]
            asgb = asg_ref[pl.ds(r * R, R), :]
            winner = winner_ref[...]
            onehot = lane == j1b
            wj = jnp.max(jnp.where(onehot, winner, -1), axis=1, keepdims=True)
            win = wj == rows_iota(r)
            changed = winner >= 0
            asg_onehot = lane == asgb
            evicted = jnp.max(
                jnp.where(jnp.logical_and(asg_onehot, changed), 1, 0),
                axis=1, keepdims=True) > 0
            newasg = jnp.where(win, j1b, jnp.where(evicted, -1, asgb))
            asg_ref[pl.ds(r * R, R), :] = jnp.where(active, newasg, asgb)
            return carry
        return pass_update

    def iter_body(t, carry):
        maxinc_ref[...] = jnp.full((1, m), _NEG_INF, jnp.float32)
        winner_ref[...] = jnp.full((1, m), -1, jnp.int32)
        lax.fori_loop(0, nb, pass_bid, 0)
        lax.fori_loop(0, nb, pass_winner, 0)
        active = t < iters
        lax.fori_loop(0, nb, make_pass_update(active), 0)
        changed = winner_ref[...] >= 0
        price = price_ref[...]
        newprice = jnp.where(changed, price + maxinc_ref[...], price)
        price_ref[...] = jnp.where(active, newprice, price)
        return carry

    def pass_output(r, carry):
        c = cost_block(r)
        asgb = asg_ref[pl.ds(r * R, R), :]
        asg_onehot = lane == asgb
        db = jnp.sum(jnp.where(asg_onehot, c, 0.0), axis=1, keepdims=True)
        dist_ref[0, pl.ds(r * R, R), :] = db
        asg_out_ref[0, pl.ds(r * R, R), :] = asgb
        return carry

    asg_ref[...] = jnp.full((n, 1), -1, jnp.int32)
    price_ref[...] = jnp.zeros((1, m), jnp.float32)
    lax.fori_loop(0, 10, iter_body, 0)
    lax.fori_loop(0, nb, pass_output, 0)


def kernel(input1, input2, eps, iters):
    x1 = input1.astype(jnp.float32)
    x2 = input2.astype(jnp.float32)
    B, n, _ = x1.shape
    m = x2.shape[1]
    x2t = jnp.transpose(x2, (0, 2, 1))
    eps_arr = jnp.asarray(eps, jnp.float32).reshape(1)
    iters_arr = jnp.asarray(iters, jnp.int32).reshape(1)

    dist3, asg3 = pl.pallas_call(
        _auction_kernel,
        grid=(B,),
        in_specs=[
            pl.BlockSpec((1, n, 3), lambda b: (b, 0, 0)),
            pl.BlockSpec((1, 3, m), lambda b: (b, 0, 0)),
            pl.BlockSpec(memory_space=pltpu.SMEM),
            pl.BlockSpec(memory_space=pltpu.SMEM),
        ],
        out_specs=[
            pl.BlockSpec((1, n, 1), lambda b: (b, 0, 0)),
            pl.BlockSpec((1, n, 1), lambda b: (b, 0, 0)),
        ],
        out_shape=[
            jax.ShapeDtypeStruct((B, n, 1), jnp.float32),
            jax.ShapeDtypeStruct((B, n, 1), jnp.int32),
        ],
        scratch_shapes=[
            pltpu.VMEM((n, 1), jnp.int32),
            pltpu.VMEM((n, 1), jnp.int32),
            pltpu.VMEM((n, 1), jnp.float32),
            pltpu.VMEM((1, m), jnp.float32),
            pltpu.VMEM((1, m), jnp.float32),
            pltpu.VMEM((1, m), jnp.int32),
        ],
        compiler_params=pltpu.CompilerParams(
            dimension_semantics=("parallel",)),
    )(x1, x2t, eps_arr, iters_arr)
    return dist3[..., 0], asg3[..., 0]
